# baseline (device time: 10891 ns/iter reference)
import jax
import jax.numpy as jnp
from jax import lax
from jax.experimental import pallas as pl
from jax.experimental.pallas import tpu as pltpu

N_DEV = 4
B, SQ, SKV = 2, 128, 128
H_PER = 4
DH = 64
D_MODEL = 512
HD = H_PER * DH
G = 64


def kernel(x, Wq, K_ext, V_ext, Wo):
    Kt = jnp.transpose(K_ext, (0, 2, 3, 1))
    Vt = jnp.transpose(V_ext, (0, 2, 3, 1))

    hbm = lambda a: pltpu.with_memory_space_constraint(
        a, pltpu.MemorySpace.HBM)

    def body(x_hbm, wq_hbm, k_hbm, v_hbm, wo_hbm, out_hbm,
             x_v, wq_v, k_v, v_v, wo_v, out_v,
             comm_ref, acc_ref, load_sems, send_sems, recv_sems):
        my = lax.axis_index("i")
        right = lax.rem(my + 1, N_DEV)
        opp = lax.rem(my + 2, N_DEV)
        left = lax.rem(my + 3, N_DEV)

        ld_x = pltpu.make_async_copy(x_hbm, x_v, load_sems.at[0])
        ld_wq = pltpu.make_async_copy(
            wq_hbm.at[:, pl.ds(my * HD, HD)], wq_v, load_sems.at[1])
        ld_k = pltpu.make_async_copy(k_hbm, k_v, load_sems.at[2])
        ld_v = pltpu.make_async_copy(v_hbm, v_v, load_sems.at[3])
        ld_wo = pltpu.make_async_copy(wo_hbm, wo_v, load_sems.at[4])
        for ld in (ld_x, ld_wq, ld_k, ld_v, ld_wo):
            ld.start()

        barrier_sem = pltpu.get_barrier_semaphore()
        for nbr in (left, right, opp):
            pl.semaphore_signal(
                barrier_sem, inc=1,
                device_id=(nbr,), device_id_type=pl.DeviceIdType.MESH,
            )

        ld_x.wait()
        ld_wq.wait()
        xf = x_v[...].reshape(B * SQ, D_MODEL).astype(jnp.bfloat16)
        qf = jnp.dot(xf, (wq_v[...] * 0.125).astype(jnp.bfloat16),
                     preferred_element_type=jnp.float32)

        pl.semaphore_wait(barrier_sem, N_DEV - 1)
        ld_k.wait()
        ld_v.wait()

        def mk(b, g, target, dst_slot, i):
            return pltpu.make_async_remote_copy(
                src_ref=comm_ref.at[0, b, pl.ds(g * G, G)],
                dst_ref=comm_ref.at[dst_slot, b, pl.ds(g * G, G)],
                send_sem=send_sems.at[b, g, i],
                recv_sem=recv_sems.at[b, g, i],
                device_id=(target,),
                device_id_type=pl.DeviceIdType.MESH,
            )

        rdmas = [[[mk(b, g, right, 3, 0), mk(b, g, left, 1, 1),
                   mk(b, g, opp, 2, 2)] for g in range(2)]
                 for b in range(B)]

        dn_qkt = (((2,), (1,)), ((0,), (0,)))
        dn_wvt = (((2,), (2,)), ((0,), (0,)))
        for b in range(B):
            qb = qf[b * SQ:(b + 1) * SQ]
            qs = jnp.stack(
                [qb[:, h * DH:(h + 1) * DH] for h in range(H_PER)]
            ).astype(jnp.bfloat16)
            kt = k_v[b].astype(jnp.bfloat16)
            vt = v_v[b].astype(jnp.bfloat16)
            for g in range(2):
                nkv = (g + 1) * G
                s = lax.dot_general(
                    qs[:, g * G:(g + 1) * G, :], kt[:, :, :nkv],
                    dn_qkt, preferred_element_type=jnp.float32)
                w = jnp.exp(s)
                w = w / jnp.sum(w, axis=-1, keepdims=True)
                ctx = lax.dot_general(
                    w.astype(jnp.bfloat16), vt[:, :, :nkv],
                    dn_wvt, preferred_element_type=jnp.float32)
                for h in range(H_PER):
                    comm_ref[0, b, g * G:(g + 1) * G,
                             h * DH:(h + 1) * DH] = ctx[h].astype(jnp.bfloat16)
                for r in rdmas[b][g]:
                    r.start()

        ld_wo.wait()
        wo_my = wo_v[pl.ds(my * HD, HD), :].astype(jnp.bfloat16)
        ctx_me = comm_ref[0].reshape(B * SQ, HD)
        acc_ref[...] = jnp.dot(ctx_me, wo_my,
                               preferred_element_type=jnp.float32)

        for i, o in ((1, 1), (0, 3), (2, 2)):
            for b in range(B):
                for g in range(2):
                    rdmas[b][g][i].wait_recv()
            src_dev = lax.rem(my + o, N_DEV)
            wo_o = wo_v[pl.ds(src_dev * HD, HD), :].astype(jnp.bfloat16)
            ctx_o = comm_ref[o].reshape(B * SQ, HD)
            acc_ref[...] = acc_ref[...] + jnp.dot(
                ctx_o, wo_o, preferred_element_type=jnp.float32)

        out_v[...] = acc_ref[...].reshape(B, SQ, D_MODEL).astype(jnp.bfloat16)
        st_out = pltpu.make_async_copy(out_v, out_hbm, load_sems.at[5])
        st_out.start()

        for b in range(B):
            for g in range(2):
                for r in rdmas[b][g]:
                    r.wait_send()
        st_out.wait()

    return pl.pallas_call(
        body,
        out_shape=jax.ShapeDtypeStruct((B, SQ, D_MODEL), jnp.bfloat16),
        in_specs=[pl.BlockSpec(memory_space=pl.ANY)] * 5,
        out_specs=pl.BlockSpec(memory_space=pltpu.MemorySpace.HBM),
        scratch_shapes=[
            pltpu.VMEM((B, SQ, D_MODEL), jnp.float32),
            pltpu.VMEM((D_MODEL, HD), jnp.float32),
            pltpu.VMEM((B, H_PER, DH, SKV), jnp.float32),
            pltpu.VMEM((B, H_PER, DH, SKV), jnp.float32),
            pltpu.VMEM((N_DEV * HD, D_MODEL), jnp.float32),
            pltpu.VMEM((B, SQ, D_MODEL), jnp.bfloat16),
            pltpu.VMEM((N_DEV, B, SQ, HD), jnp.bfloat16),
            pltpu.VMEM((B * SQ, D_MODEL), jnp.float32),
            pltpu.SemaphoreType.DMA((6,)),
            pltpu.SemaphoreType.DMA((B, 2, 3)),
            pltpu.SemaphoreType.DMA((B, 2, 3)),
        ],
        compiler_params=pltpu.CompilerParams(collective_id=0),
    )(hbm(x), hbm(Wq), hbm(Kt), hbm(Vt), hbm(Wo))


# device time: 10885 ns/iter; 1.0006x vs baseline; 1.0006x over previous
import jax
import jax.numpy as jnp
from jax import lax
from jax.experimental import pallas as pl
from jax.experimental.pallas import tpu as pltpu

N_DEV = 4
B, SQ, SKV = 2, 128, 128
H_PER = 4
DH = 64
D_MODEL = 512
HD = H_PER * DH


def kernel(x, Wq, K_ext, V_ext, Wo):
    Kt = jnp.transpose(K_ext, (0, 2, 3, 1))
    Vt = jnp.transpose(V_ext, (0, 2, 3, 1))

    hbm = lambda a: pltpu.with_memory_space_constraint(
        a, pltpu.MemorySpace.HBM)

    def body(x_hbm, wq_hbm, k_hbm, v_hbm, wo_hbm, out_hbm,
             x_v, wq_v, k_v, v_v, wo_v, out_v,
             comm_ref, acc_ref, load_sems, send_sems, recv_sems):
        my = lax.axis_index("i")
        right = lax.rem(my + 1, N_DEV)
        opp = lax.rem(my + 2, N_DEV)
        left = lax.rem(my + 3, N_DEV)

        ld_x = pltpu.make_async_copy(x_hbm, x_v, load_sems.at[0])
        ld_wq = pltpu.make_async_copy(
            wq_hbm.at[:, pl.ds(my * HD, HD)], wq_v, load_sems.at[1])
        ld_k = pltpu.make_async_copy(k_hbm, k_v, load_sems.at[2])
        ld_v = pltpu.make_async_copy(v_hbm, v_v, load_sems.at[3])
        ld_wo = [
            pltpu.make_async_copy(
                wo_hbm.at[pl.ds(lax.rem(my + o, N_DEV) * HD, HD), :],
                wo_v.at[pl.ds(o * HD, HD), :],
                load_sems.at[4 + o])
            for o in range(N_DEV)
        ]
        for ld in (ld_x, ld_wq, ld_k, ld_v, *ld_wo):
            ld.start()

        barrier_sem = pltpu.get_barrier_semaphore()
        for nbr in (left, right, opp):
            pl.semaphore_signal(
                barrier_sem, inc=1,
                device_id=(nbr,), device_id_type=pl.DeviceIdType.MESH,
            )

        row_blk = lax.broadcasted_iota(jnp.int32, (SQ, SKV), 0) // 64
        col_blk = lax.broadcasted_iota(jnp.int32, (SQ, SKV), 1) // 64
        mask = (col_blk <= row_blk)[None]

        ld_x.wait()
        ld_wq.wait()
        xf = x_v[...].reshape(B * SQ, D_MODEL).astype(jnp.bfloat16)
        qf = jnp.dot(xf, (wq_v[...] * 0.125).astype(jnp.bfloat16),
                     preferred_element_type=jnp.float32)

        pl.semaphore_wait(barrier_sem, N_DEV - 1)
        ld_k.wait()
        ld_v.wait()

        def mk(b, target, dst_slot, i):
            return pltpu.make_async_remote_copy(
                src_ref=comm_ref.at[0, b],
                dst_ref=comm_ref.at[dst_slot, b],
                send_sem=send_sems.at[b, i],
                recv_sem=recv_sems.at[b, i],
                device_id=(target,),
                device_id_type=pl.DeviceIdType.MESH,
            )

        rdmas = [[mk(b, right, 3, 0), mk(b, left, 1, 1), mk(b, opp, 2, 2)]
                 for b in range(B)]

        dn_qkt = (((2,), (1,)), ((0,), (0,)))
        dn_wvt = (((2,), (2,)), ((0,), (0,)))
        for b in range(B):
            qb = qf[b * SQ:(b + 1) * SQ]
            qs = jnp.stack(
                [qb[:, h * DH:(h + 1) * DH] for h in range(H_PER)]
            ).astype(jnp.bfloat16)
            kt = k_v[b].astype(jnp.bfloat16)
            vt = v_v[b].astype(jnp.bfloat16)
            s = lax.dot_general(qs, kt, dn_qkt,
                                preferred_element_type=jnp.float32)
            w = jnp.exp(jnp.where(mask, s, -1e9))
            w = w / jnp.sum(w, axis=-1, keepdims=True)
            ctx = lax.dot_general(w.astype(jnp.bfloat16), vt, dn_wvt,
                                  preferred_element_type=jnp.float32)
            for h in range(H_PER):
                comm_ref[0, b, :, h * DH:(h + 1) * DH] = (
                    ctx[h].astype(jnp.bfloat16))
            for r in rdmas[b]:
                r.start()

        ld_wo[0].wait()
        wo_my = wo_v[:HD, :].astype(jnp.bfloat16)
        ctx_me = comm_ref[0].reshape(B * SQ, HD)
        acc_ref[...] = jnp.dot(ctx_me, wo_my,
                               preferred_element_type=jnp.float32)

        for b in range(B):
            for i in range(3):
                rdmas[b][i].wait_recv()
        for o in range(1, N_DEV):
            ld_wo[o].wait()
        ctx_peers = jnp.concatenate(
            [comm_ref[o].reshape(B * SQ, HD) for o in range(1, N_DEV)],
            axis=1)
        out = acc_ref[...] + jnp.dot(
            ctx_peers, wo_v[HD:, :].astype(jnp.bfloat16),
            preferred_element_type=jnp.float32)
        out_v[...] = out.reshape(B, SQ, D_MODEL).astype(jnp.bfloat16)
        st_out = pltpu.make_async_copy(out_v, out_hbm, load_sems.at[8])
        st_out.start()

        for b in range(B):
            for r in rdmas[b]:
                r.wait_send()
        st_out.wait()

    return pl.pallas_call(
        body,
        out_shape=jax.ShapeDtypeStruct((B, SQ, D_MODEL), jnp.bfloat16),
        in_specs=[pl.BlockSpec(memory_space=pl.ANY)] * 5,
        out_specs=pl.BlockSpec(memory_space=pltpu.MemorySpace.HBM),
        scratch_shapes=[
            pltpu.VMEM((B, SQ, D_MODEL), jnp.float32),
            pltpu.VMEM((D_MODEL, HD), jnp.float32),
            pltpu.VMEM((B, H_PER, DH, SKV), jnp.float32),
            pltpu.VMEM((B, H_PER, DH, SKV), jnp.float32),
            pltpu.VMEM((N_DEV * HD, D_MODEL), jnp.float32),
            pltpu.VMEM((B, SQ, D_MODEL), jnp.bfloat16),
            pltpu.VMEM((N_DEV, B, SQ, HD), jnp.bfloat16),
            pltpu.VMEM((B * SQ, D_MODEL), jnp.float32),
            pltpu.SemaphoreType.DMA((9,)),
            pltpu.SemaphoreType.DMA((B, 3)),
            pltpu.SemaphoreType.DMA((B, 3)),
        ],
        compiler_params=pltpu.CompilerParams(collective_id=0),
    )(hbm(x), hbm(Wq), hbm(Kt), hbm(Vt), hbm(Wo))


# device time: 10660 ns/iter; 1.0217x vs baseline; 1.0211x over previous
import jax
import jax.numpy as jnp
from jax import lax
from jax.experimental import pallas as pl
from jax.experimental.pallas import tpu as pltpu

N_DEV = 4
B, SQ, SKV = 2, 128, 128
H_PER = 4
DH = 64
D_MODEL = 512
HD = H_PER * DH


def kernel(x, Wq, K_ext, V_ext, Wo):
    Kt = jnp.transpose(K_ext, (0, 2, 3, 1))
    Vt = jnp.transpose(V_ext, (0, 2, 3, 1))

    hbm = lambda a: pltpu.with_memory_space_constraint(
        a, pltpu.MemorySpace.HBM)

    def body(x_hbm, wq_hbm, k_hbm, v_hbm, wo_hbm, out_hbm,
             x_v, wq_v, k_v, v_v, wo_v, out_v,
             comm_ref, acc_ref, load_sems, send_sems, recv_sems):
        my = lax.axis_index("i")
        right = lax.rem(my + 1, N_DEV)
        opp = lax.rem(my + 2, N_DEV)
        left = lax.rem(my + 3, N_DEV)

        ld_x = pltpu.make_async_copy(x_hbm, x_v, load_sems.at[0])
        ld_wq = pltpu.make_async_copy(
            wq_hbm.at[:, pl.ds(my * HD, HD)], wq_v, load_sems.at[1])
        ld_k = pltpu.make_async_copy(k_hbm, k_v, load_sems.at[2])
        ld_v = pltpu.make_async_copy(v_hbm, v_v, load_sems.at[3])
        ld_wo = pltpu.make_async_copy(wo_hbm, wo_v, load_sems.at[4])
        for ld in (ld_x, ld_wq, ld_k, ld_v, ld_wo):
            ld.start()

        barrier_sem = pltpu.get_barrier_semaphore()
        for nbr in (left, right, opp):
            pl.semaphore_signal(
                barrier_sem, inc=1,
                device_id=(nbr,), device_id_type=pl.DeviceIdType.MESH,
            )

        row_blk = lax.broadcasted_iota(jnp.int32, (SQ, SKV), 0) // 64
        col_blk = lax.broadcasted_iota(jnp.int32, (SQ, SKV), 1) // 64
        mask = (col_blk <= row_blk)[None]

        ld_x.wait()
        ld_wq.wait()
        xf = x_v[...].reshape(B * SQ, D_MODEL).astype(jnp.bfloat16)
        qf = jnp.dot(xf, (wq_v[...] * 0.125).astype(jnp.bfloat16),
                     preferred_element_type=jnp.float32)

        pl.semaphore_wait(barrier_sem, N_DEV - 1)
        ld_k.wait()
        ld_v.wait()

        def mk(b, target, dst_slot, i):
            return pltpu.make_async_remote_copy(
                src_ref=comm_ref.at[0, b],
                dst_ref=comm_ref.at[dst_slot, b],
                send_sem=send_sems.at[b, i],
                recv_sem=recv_sems.at[b, i],
                device_id=(target,),
                device_id_type=pl.DeviceIdType.MESH,
            )

        rdmas = [[mk(b, right, 3, 0), mk(b, left, 1, 1), mk(b, opp, 2, 2)]
                 for b in range(B)]

        dn_qkt = (((2,), (1,)), ((0,), (0,)))
        dn_wvt = (((2,), (2,)), ((0,), (0,)))
        for b in range(B):
            qb = qf[b * SQ:(b + 1) * SQ]
            qs = jnp.stack(
                [qb[:, h * DH:(h + 1) * DH] for h in range(H_PER)]
            ).astype(jnp.bfloat16)
            kt = k_v[b].astype(jnp.bfloat16)
            vt = v_v[b].astype(jnp.bfloat16)
            s = lax.dot_general(qs, kt, dn_qkt,
                                preferred_element_type=jnp.float32)
            w = jnp.exp(jnp.where(mask, s, -1e9))
            w = w / jnp.sum(w, axis=-1, keepdims=True)
            ctx = lax.dot_general(w.astype(jnp.bfloat16), vt, dn_wvt,
                                  preferred_element_type=jnp.float32)
            for h in range(H_PER):
                comm_ref[0, b, :, h * DH:(h + 1) * DH] = (
                    ctx[h].astype(jnp.bfloat16))
            for r in rdmas[b]:
                r.start()

        ld_wo.wait()
        wo_my = wo_v[pl.ds(my * HD, HD), :].astype(jnp.bfloat16)
        ctx_me = comm_ref[0].reshape(B * SQ, HD)
        acc_ref[...] = jnp.dot(ctx_me, wo_my,
                               preferred_element_type=jnp.float32)

        for i, o in ((1, 1), (0, 3), (2, 2)):
            for b in range(B):
                rdmas[b][i].wait_recv()
            src_dev = lax.rem(my + o, N_DEV)
            wo_o = wo_v[pl.ds(src_dev * HD, HD), :].astype(jnp.bfloat16)
            ctx_o = comm_ref[o].reshape(B * SQ, HD)
            acc_ref[...] = acc_ref[...] + jnp.dot(
                ctx_o, wo_o, preferred_element_type=jnp.float32)

        out_v[...] = acc_ref[...].reshape(B, SQ, D_MODEL).astype(jnp.bfloat16)
        st_out = pltpu.make_async_copy(out_v, out_hbm, load_sems.at[5])
        st_out.start()

        for b in range(B):
            for r in rdmas[b]:
                r.wait_send()
        st_out.wait()

    return pl.pallas_call(
        body,
        out_shape=jax.ShapeDtypeStruct((B, SQ, D_MODEL), jnp.bfloat16),
        in_specs=[pl.BlockSpec(memory_space=pl.ANY)] * 5,
        out_specs=pl.BlockSpec(memory_space=pltpu.MemorySpace.HBM),
        scratch_shapes=[
            pltpu.VMEM((B, SQ, D_MODEL), jnp.float32),
            pltpu.VMEM((D_MODEL, HD), jnp.float32),
            pltpu.VMEM((B, H_PER, DH, SKV), jnp.float32),
            pltpu.VMEM((B, H_PER, DH, SKV), jnp.float32),
            pltpu.VMEM((N_DEV * HD, D_MODEL), jnp.float32),
            pltpu.VMEM((B, SQ, D_MODEL), jnp.bfloat16),
            pltpu.VMEM((N_DEV, B, SQ, HD), jnp.bfloat16),
            pltpu.VMEM((B * SQ, D_MODEL), jnp.float32),
            pltpu.SemaphoreType.DMA((6,)),
            pltpu.SemaphoreType.DMA((B, 3)),
            pltpu.SemaphoreType.DMA((B, 3)),
        ],
        compiler_params=pltpu.CompilerParams(collective_id=0),
    )(hbm(x), hbm(Wq), hbm(Kt), hbm(Vt), hbm(Wo))
